# baseline (speedup 1.0000x reference)
import functools

import jax
import jax.numpy as jnp
from jax import lax
from jax.experimental import pallas as pl
from jax.experimental.pallas import tpu as pltpu
from jax.experimental.pallas import tpu_sc as plsc

N_NODES = 10000
N_EDGES = 320000
IN_DIM = 128
OUT_DIM = 128

NC = 2
NS = 16
CHUNK = 128
CHUNKS_PER_TILE = 80
PAD_EDGES = NC * NS * CHUNKS_PER_TILE * CHUNK
ROWS_PER_TILE = 640
ACC_ROWS = NS * ROWS_PER_TILE


def _mm_body(x_ref, w_ref, o_ref):
    o_ref[...] = jnp.dot(x_ref[...], w_ref[...], preferred_element_type=jnp.float32)


def _matmul(x, w):
    grid = 10
    blk = N_NODES // grid
    return pl.pallas_call(
        _mm_body,
        grid=(grid,),
        in_specs=[
            pl.BlockSpec((blk, IN_DIM), lambda i: (i, 0)),
            pl.BlockSpec((IN_DIM, OUT_DIM), lambda i: (0, 0)),
        ],
        out_specs=pl.BlockSpec((blk, OUT_DIM), lambda i: (i, 0)),
        out_shape=jax.ShapeDtypeStruct((N_NODES, OUT_DIM), jnp.float32),
    )(x, w)


_sc_mesh = plsc.VectorSubcoreMesh(
    core_axis_name="c", subcore_axis_name="s", num_cores=NC, num_subcores=NS
)


@functools.partial(
    pl.kernel,
    out_type=jax.ShapeDtypeStruct((NC * ACC_ROWS, OUT_DIM), jnp.float32),
    mesh=_sc_mesh,
    scratch_types=[
        pltpu.VMEM((CHUNK,), jnp.int32),
        pltpu.VMEM((CHUNK,), jnp.int32),
        pltpu.VMEM((CHUNK, OUT_DIM), jnp.float32),
        pltpu.VMEM((CHUNK,), jnp.int32),
        pltpu.VMEM((CHUNK,), jnp.int32),
        pltpu.VMEM((CHUNK, OUT_DIM), jnp.float32),
        pltpu.VMEM_SHARED((ACC_ROWS, OUT_DIM), jnp.float32),
        pltpu.SemaphoreType.DMA,
        pltpu.SemaphoreType.DMA,
    ],
)
def _sc_aggregate(src_hbm, dst_hbm, h_hbm, z_hbm, out_hbm,
                  src_a, dst_a, rows_a, src_b, dst_b, rows_b, acc,
                  sem_a, sem_b):
    c = lax.axis_index("c")
    s = lax.axis_index("s")
    wid = c * NS + s

    pltpu.sync_copy(z_hbm, rows_a)
    for k in range(ROWS_PER_TILE // CHUNK):
        pltpu.sync_copy(
            rows_a, acc.at[pl.ds(s * ROWS_PER_TILE + k * CHUNK, CHUNK)]
        )
    plsc.subcore_barrier()

    base = wid * (CHUNKS_PER_TILE * CHUNK)

    def load_idx(chunk, src_v, dst_v):
        e0 = base + chunk * CHUNK
        pltpu.sync_copy(src_hbm.at[pl.ds(e0, CHUNK)], src_v)
        pltpu.sync_copy(dst_hbm.at[pl.ds(e0, CHUNK)], dst_v)

    def gather_start(src_v, rows_v, sem):
        pltpu.async_copy(h_hbm.at[src_v], rows_v, sem)

    def gather_wait(src_v, rows_v, sem):
        pltpu.make_async_copy(h_hbm.at[src_v], rows_v, sem).wait()

    def scatter(rows_v, dst_v):
        pltpu.sync_copy(rows_v, acc.at[dst_v], add=True)

    def pair_body(k, carry):
        c0 = 2 * k
        load_idx(c0, src_a, dst_a)
        load_idx(c0 + 1, src_b, dst_b)
        gather_start(src_a, rows_a, sem_a)
        gather_start(src_b, rows_b, sem_b)
        gather_wait(src_a, rows_a, sem_a)
        scatter(rows_a, dst_a)
        gather_wait(src_b, rows_b, sem_b)
        scatter(rows_b, dst_b)
        return carry

    lax.fori_loop(0, CHUNKS_PER_TILE // 2, pair_body, 0)
    plsc.subcore_barrier()

    pltpu.sync_copy(
        acc.at[pl.ds(s * ROWS_PER_TILE, ROWS_PER_TILE)],
        out_hbm.at[pl.ds(c * ACC_ROWS + s * ROWS_PER_TILE, ROWS_PER_TILE)],
    )


def _combine_body(p_ref, o_ref):
    o_ref[...] = jnp.maximum(p_ref[0] + p_ref[1], 0.0)


def _combine(partials):
    grid = 10
    blk = N_NODES // grid
    return pl.pallas_call(
        _combine_body,
        grid=(grid,),
        in_specs=[pl.BlockSpec((NC, blk, OUT_DIM), lambda i: (0, i, 0))],
        out_specs=pl.BlockSpec((blk, OUT_DIM), lambda i: (i, 0)),
        out_shape=jax.ShapeDtypeStruct((N_NODES, OUT_DIM), jnp.float32),
    )(partials)


def kernel(x, edge_index, W):
    ei = edge_index.astype(jnp.int32)
    dst = ei[0]
    src = ei[1]
    pad = PAD_EDGES - N_EDGES
    src_p = jnp.concatenate([src, jnp.zeros((pad,), jnp.int32)])
    dump_rows = N_NODES + jnp.arange(pad, dtype=jnp.int32) % (ACC_ROWS - N_NODES)
    dst_p = jnp.concatenate([dst, dump_rows])
    zeros_rows = jnp.zeros((CHUNK, OUT_DIM), jnp.float32)

    h = _matmul(x, W)
    partials = _sc_aggregate(src_p, dst_p, h, zeros_rows)
    p2 = partials.reshape(NC, ACC_ROWS, OUT_DIM)[:, :N_NODES, :]
    return _combine(p2)


# speedup vs baseline: 1.3574x; 1.3574x over previous
import functools

import jax
import jax.numpy as jnp
from jax import lax
from jax.experimental import pallas as pl
from jax.experimental.pallas import tpu as pltpu
from jax.experimental.pallas import tpu_sc as plsc

N_NODES = 10000
N_EDGES = 320000
IN_DIM = 128
OUT_DIM = 128

NC = 2
NS = 16
CHUNK = 128
CHUNKS_PER_TILE = 79
PAD_EDGES = NC * NS * CHUNKS_PER_TILE * CHUNK
ROWS_PER_TILE = 640
ACC_ROWS = NS * ROWS_PER_TILE


def _mm_body(x_ref, w_ref, o_ref):
    o_ref[...] = jnp.dot(x_ref[...], w_ref[...], preferred_element_type=jnp.float32)


def _matmul(x, w):
    grid = 10
    blk = N_NODES // grid
    return pl.pallas_call(
        _mm_body,
        grid=(grid,),
        in_specs=[
            pl.BlockSpec((blk, IN_DIM), lambda i: (i, 0)),
            pl.BlockSpec((IN_DIM, OUT_DIM), lambda i: (0, 0)),
        ],
        out_specs=pl.BlockSpec((blk, OUT_DIM), lambda i: (i, 0)),
        out_shape=jax.ShapeDtypeStruct((N_NODES, OUT_DIM), jnp.float32),
    )(x, w)


_sc_mesh = plsc.VectorSubcoreMesh(
    core_axis_name="c", subcore_axis_name="s", num_cores=NC, num_subcores=NS
)


@functools.partial(
    pl.kernel,
    out_type=jax.ShapeDtypeStruct((NC * ACC_ROWS, OUT_DIM), jnp.float32),
    mesh=_sc_mesh,
    scratch_types=[
        pltpu.VMEM((CHUNK,), jnp.int32),
        pltpu.VMEM((CHUNK,), jnp.int32),
        pltpu.VMEM((CHUNK, OUT_DIM), jnp.float32),
        pltpu.VMEM((CHUNK,), jnp.int32),
        pltpu.VMEM((CHUNK,), jnp.int32),
        pltpu.VMEM((CHUNK, OUT_DIM), jnp.float32),
        pltpu.VMEM_SHARED((ACC_ROWS, OUT_DIM), jnp.float32),
        pltpu.SemaphoreType.DMA,
        pltpu.SemaphoreType.DMA,
    ],
)
def _sc_aggregate(src_hbm, dst_hbm, h_hbm, z_hbm, out_hbm,
                  src_a, dst_a, rows_a, src_b, dst_b, rows_b, acc,
                  sem_a, sem_b):
    c = lax.axis_index("c")
    s = lax.axis_index("s")
    wid = c * NS + s

    pltpu.sync_copy(z_hbm, rows_a)
    for k in range(ROWS_PER_TILE // CHUNK):
        pltpu.sync_copy(
            rows_a, acc.at[pl.ds(s * ROWS_PER_TILE + k * CHUNK, CHUNK)]
        )
    plsc.subcore_barrier()

    base = wid * (CHUNKS_PER_TILE * CHUNK)

    def load_idx(chunk, src_v, dst_v):
        e0 = base + chunk * CHUNK
        pltpu.sync_copy(src_hbm.at[pl.ds(e0, CHUNK)], src_v)
        pltpu.sync_copy(dst_hbm.at[pl.ds(e0, CHUNK)], dst_v)

    def gather_start(src_v, rows_v, sem):
        pltpu.async_copy(h_hbm.at[src_v], rows_v, sem)

    def gather_wait(src_v, rows_v, sem):
        pltpu.make_async_copy(h_hbm.at[src_v], rows_v, sem).wait()

    def scatter(rows_v, dst_v):
        pltpu.sync_copy(rows_v, acc.at[dst_v], add=True)

    def body(i, carry):
        load_idx(i, src_a, dst_a)
        gather_start(src_a, rows_a, sem_a)
        gather_wait(src_a, rows_a, sem_a)
        scatter(rows_a, dst_a)
        return carry

    lax.fori_loop(0, CHUNKS_PER_TILE, body, 0)
    plsc.subcore_barrier()

    pltpu.sync_copy(
        acc.at[pl.ds(s * ROWS_PER_TILE, ROWS_PER_TILE)],
        out_hbm.at[pl.ds(c * ACC_ROWS + s * ROWS_PER_TILE, ROWS_PER_TILE)],
    )


def _combine_body(p_ref, o_ref):
    o_ref[...] = jnp.maximum(p_ref[0] + p_ref[1], 0.0)


def _combine(partials):
    grid = 10
    blk = N_NODES // grid
    return pl.pallas_call(
        _combine_body,
        grid=(grid,),
        in_specs=[pl.BlockSpec((NC, blk, OUT_DIM), lambda i: (0, i, 0))],
        out_specs=pl.BlockSpec((blk, OUT_DIM), lambda i: (i, 0)),
        out_shape=jax.ShapeDtypeStruct((N_NODES, OUT_DIM), jnp.float32),
    )(partials)


def kernel(x, edge_index, W):
    ei = edge_index.astype(jnp.int32)
    dst = ei[0]
    src = ei[1]
    pad = PAD_EDGES - N_EDGES
    src_p = jnp.concatenate([src, jnp.zeros((pad,), jnp.int32)])
    dump_rows = N_NODES + jnp.arange(pad, dtype=jnp.int32) % (ACC_ROWS - N_NODES)
    dst_p = jnp.concatenate([dst, dump_rows])
    zeros_rows = jnp.zeros((CHUNK, OUT_DIM), jnp.float32)

    h = _matmul(x, W)
    partials = _sc_aggregate(src_p, dst_p, h, zeros_rows)
    p2 = partials.reshape(NC, ACC_ROWS, OUT_DIM)[:, :N_NODES, :]
    return _combine(p2)


# speedup vs baseline: 1.6261x; 1.1980x over previous
import functools

import jax
import jax.numpy as jnp
from jax import lax
from jax.experimental import pallas as pl
from jax.experimental.pallas import tpu as pltpu
from jax.experimental.pallas import tpu_sc as plsc

N_NODES = 10000
N_EDGES = 320000
IN_DIM = 128
OUT_DIM = 128

NC = 2
NS = 16
CHUNK = 128
CHUNKS_PER_TILE = 79
PAD_EDGES = NC * NS * CHUNKS_PER_TILE * CHUNK
ROWS_PER_TILE = 640
ACC_ROWS = NS * ROWS_PER_TILE


def _mm_body(x_ref, w_ref, o_ref):
    o_ref[...] = jnp.dot(x_ref[...], w_ref[...], preferred_element_type=jnp.float32)


def _matmul(x, w):
    grid = 10
    blk = N_NODES // grid
    return pl.pallas_call(
        _mm_body,
        grid=(grid,),
        in_specs=[
            pl.BlockSpec((blk, IN_DIM), lambda i: (i, 0)),
            pl.BlockSpec((IN_DIM, OUT_DIM), lambda i: (0, 0)),
        ],
        out_specs=pl.BlockSpec((blk, OUT_DIM), lambda i: (i, 0)),
        out_shape=jax.ShapeDtypeStruct((N_NODES, OUT_DIM), jnp.float32),
    )(x, w)


_sc_mesh = plsc.VectorSubcoreMesh(
    core_axis_name="c", subcore_axis_name="s", num_cores=NC, num_subcores=NS
)


@functools.partial(
    pl.kernel,
    out_type=jax.ShapeDtypeStruct((NC * ACC_ROWS, OUT_DIM), jnp.float32),
    mesh=_sc_mesh,
    scratch_types=[
        pltpu.VMEM((CHUNKS_PER_TILE, 2, CHUNK), jnp.int32),
        pltpu.VMEM((CHUNK, OUT_DIM), jnp.float32),
        pltpu.VMEM_SHARED((ACC_ROWS, OUT_DIM), jnp.float32),
        pltpu.SemaphoreType.DMA,
    ],
)
def _sc_aggregate(epk_hbm, h_hbm, z_hbm, out_hbm, idx_v, rows_a, acc, sem_a):
    c = lax.axis_index("c")
    s = lax.axis_index("s")
    wid = c * NS + s

    pltpu.sync_copy(epk_hbm.at[wid], idx_v)
    pltpu.sync_copy(z_hbm, rows_a)
    for k in range(ROWS_PER_TILE // CHUNK):
        pltpu.sync_copy(
            rows_a, acc.at[pl.ds(s * ROWS_PER_TILE + k * CHUNK, CHUNK)]
        )
    plsc.subcore_barrier()

    def body(i, carry):
        pltpu.async_copy(h_hbm.at[idx_v.at[i, 0]], rows_a, sem_a).wait()
        pltpu.sync_copy(rows_a, acc.at[idx_v.at[i, 1]], add=True)
        return carry

    lax.fori_loop(0, CHUNKS_PER_TILE, body, 0)
    plsc.subcore_barrier()

    pltpu.sync_copy(
        acc.at[pl.ds(s * ROWS_PER_TILE, ROWS_PER_TILE)],
        out_hbm.at[pl.ds(c * ACC_ROWS + s * ROWS_PER_TILE, ROWS_PER_TILE)],
    )


def _combine_body(p_ref, o_ref):
    o_ref[...] = jnp.maximum(p_ref[0] + p_ref[1], 0.0)


def _combine(partials):
    grid = 10
    blk = N_NODES // grid
    return pl.pallas_call(
        _combine_body,
        grid=(grid,),
        in_specs=[pl.BlockSpec((NC, blk, OUT_DIM), lambda i: (0, i, 0))],
        out_specs=pl.BlockSpec((blk, OUT_DIM), lambda i: (i, 0)),
        out_shape=jax.ShapeDtypeStruct((N_NODES, OUT_DIM), jnp.float32),
    )(partials)


def kernel(x, edge_index, W):
    ei = edge_index.astype(jnp.int32)
    dst = ei[0]
    src = ei[1]
    pad = PAD_EDGES - N_EDGES
    src_p = jnp.concatenate([src, jnp.zeros((pad,), jnp.int32)])
    dump_rows = N_NODES + jnp.arange(pad, dtype=jnp.int32) % (ACC_ROWS - N_NODES)
    dst_p = jnp.concatenate([dst, dump_rows])
    epk = jnp.stack(
        [src_p.reshape(NC * NS, CHUNKS_PER_TILE, CHUNK),
         dst_p.reshape(NC * NS, CHUNKS_PER_TILE, CHUNK)],
        axis=2,
    )
    zeros_rows = jnp.zeros((CHUNK, OUT_DIM), jnp.float32)

    h = _matmul(x, W)
    partials = _sc_aggregate(epk, h, zeros_rows)
    p2 = partials.reshape(NC, ACC_ROWS, OUT_DIM)[:, :N_NODES, :]
    return _combine(p2)
